# Initial kernel scaffold; baseline (speedup 1.0000x reference)
import functools

import jax
import jax.numpy as jnp
from jax import lax
from jax.experimental import pallas as pl
from jax.experimental.pallas import tpu as pltpu
from jax.experimental.pallas import tpu_sc as plsc

KK = 3
MM = 10000
EE = 320000
FIN = 128
NB = 4
FF = FIN * NB
W = 128
NCH = FF // W
NS = 16
CH_PER_CORE = NCH // 2

EPT = EE // NS
EB = 80
NBLK = EPT // EB
RPT = MM // NS
RB = 125
NRB = RPT // RB
LANES = 16


def _sc_body(x0f, rows_h, cols_h, vals_h, out_h,
             y_sp, cols_raw, cols_adj, rows_v, vals_v, gbuf, ybuf, gsem):
    cid = lax.axis_index("c")
    sid = lax.axis_index("s")
    ebase = sid * EPT
    rbase = sid * RPT

    for ci in range(CH_PER_CORE):
        cg = cid * CH_PER_CORE + ci
        cgM = cg * MM
        for k in range(KK):
            for i in range(NRB):
                r0 = rbase + i * RB
                pltpu.sync_copy(x0f.at[pl.ds(cgM + r0, RB)], ybuf)
                pltpu.sync_copy(ybuf, y_sp.at[pl.ds(r0, RB)])
            plsc.subcore_barrier()

            def blk_body(b, carry, k=k, cgM=cgM, ebase=ebase):
                eoff = ebase + b * EB
                pltpu.sync_copy(cols_h.at[k, pl.ds(eoff, EB)], cols_raw)
                pltpu.sync_copy(rows_h.at[k, pl.ds(eoff, EB)], rows_v)
                pltpu.sync_copy(vals_h.at[k, pl.ds(eoff, EB)], vals_v)
                for g in range(EB // LANES):
                    sl = pl.ds(g * LANES, LANES)
                    cols_adj[sl] = cols_raw[sl] + cgM
                pltpu.async_copy(x0f.at[cols_adj], gbuf, gsem).wait()

                def edge_body(e, c2):
                    v = vals_v[e]
                    for j in range(W // LANES):
                        sl = pl.ds(j * LANES, LANES)
                        gbuf[e, sl] = gbuf[e, sl] * v
                    return c2
                lax.fori_loop(0, EB, edge_body, 0)

                pltpu.sync_copy(gbuf, y_sp.at[rows_v], add=True)
                return carry
            lax.fori_loop(0, NBLK, blk_body, 0)
            plsc.subcore_barrier()

            for i in range(NRB):
                r0 = rbase + i * RB
                pltpu.sync_copy(y_sp.at[pl.ds(r0, RB)], ybuf)

                def relu_body(r, c2):
                    for j in range(W // LANES):
                        sl = pl.ds(j * LANES, LANES)
                        ybuf[r, sl] = jnp.maximum(ybuf[r, sl], 0.0)
                    return c2
                lax.fori_loop(0, RB, relu_body, 0)
                pltpu.sync_copy(ybuf, out_h.at[k, pl.ds(r0, RB), pl.ds(cg * W, W)])
            plsc.subcore_barrier()


_sc_call = functools.partial(
    pl.kernel,
    mesh=plsc.VectorSubcoreMesh(core_axis_name="c", subcore_axis_name="s"),
    out_type=jax.ShapeDtypeStruct((KK, MM, FF), jnp.float32),
    scratch_types=[
        pltpu.VMEM_SHARED((MM, W), jnp.float32),
        pltpu.VMEM((EB,), jnp.int32),
        pltpu.VMEM((EB,), jnp.int32),
        pltpu.VMEM((EB,), jnp.int32),
        pltpu.VMEM((EB,), jnp.float32),
        pltpu.VMEM((EB, W), jnp.float32),
        pltpu.VMEM((RB, W), jnp.float32),
        pltpu.SemaphoreType.DMA,
    ],
)(_sc_body)


@jax.jit
def kernel(x, L_rows, L_cols, L_vals):
    x0 = jnp.transpose(x, (1, 2, 0)).reshape(MM, FF)
    x0f = x0.reshape(MM, NCH, W).transpose(1, 0, 2).reshape(NCH * MM, W)
    out = _sc_call(x0f,
                   L_rows.astype(jnp.int32),
                   L_cols.astype(jnp.int32),
                   L_vals)
    return jnp.transpose(out.reshape(KK, MM, FIN, NB), (3, 1, 2, 0)).reshape(
        NB, MM, FIN * KK)


# speedup vs baseline: 1.6751x; 1.6751x over previous
import functools

import jax
import jax.numpy as jnp
from jax import lax
from jax.experimental import pallas as pl
from jax.experimental.pallas import tpu as pltpu
from jax.experimental.pallas import tpu_sc as plsc

KK = 3
MM = 10000
EE = 320000
FIN = 128
NB = 4
FF = FIN * NB
W = 128
NCH = FF // W
NS = 16
CH_PER_CORE = NCH // 2

EPT = EE // NS
EB = 80
NBLK = EPT // EB
RB = 200
NRB = MM // RB
RB_ITERS = -(-NRB // NS)
LANES = 16


def _sc_body(x0f, rows_h, cols_h, vals_h, out_h,
             y_sp, cols_raw, cols_adj, rows_v, vals_v, gbuf, ybuf, gsem):
    cid = lax.axis_index("c")
    sid = lax.axis_index("s")
    ebase = sid * EPT

    for ci in range(CH_PER_CORE):
        cg = cid * CH_PER_CORE + ci
        cgM = cg * MM
        for k in range(KK):
            for i in range(RB_ITERS):
                bid = sid + NS * i
                @pl.when(bid < NRB)
                def _():
                    r0 = pl.multiple_of(bid * RB, 8)
                    pltpu.sync_copy(x0f.at[pl.ds(cgM + r0, RB)], ybuf)
                    pltpu.sync_copy(ybuf, y_sp.at[pl.ds(r0, RB)])
            plsc.subcore_barrier()

            def blk_body(b, carry, k=k, cgM=cgM, ebase=ebase):
                eoff = pl.multiple_of(k * EE + ebase + b * EB, 8)
                pltpu.sync_copy(cols_h.at[pl.ds(eoff, EB)], cols_raw)
                pltpu.sync_copy(rows_h.at[pl.ds(eoff, EB)], rows_v)
                pltpu.sync_copy(vals_h.at[pl.ds(eoff, EB)], vals_v)
                for g in range(EB // LANES):
                    sl = pl.ds(g * LANES, LANES)
                    cols_adj[sl] = cols_raw[sl] + cgM
                pltpu.async_copy(x0f.at[cols_adj], gbuf, gsem).wait()

                def grp_body(g, c2):
                    vv = vals_v[pl.ds(g * LANES, LANES)]
                    for e16 in range(LANES):
                        v = vv[e16]
                        row = g * LANES + e16
                        for j in range(W // LANES):
                            sl = pl.ds(j * LANES, LANES)
                            gbuf[row, sl] = gbuf[row, sl] * v
                    return c2
                lax.fori_loop(0, EB // LANES, grp_body, 0)

                pltpu.sync_copy(gbuf, y_sp.at[rows_v], add=True)
                return carry
            lax.fori_loop(0, NBLK, blk_body, 0)
            plsc.subcore_barrier()

            for i in range(RB_ITERS):
                bid = sid + NS * i
                @pl.when(bid < NRB)
                def _():
                    r0 = pl.multiple_of(bid * RB, 8)
                    pltpu.sync_copy(y_sp.at[pl.ds(r0, RB)], ybuf)

                    def relu_body(r, c2):
                        for j in range(W // LANES):
                            sl = pl.ds(j * LANES, LANES)
                            ybuf[r, sl] = jnp.maximum(ybuf[r, sl], 0.0)
                        return c2
                    lax.fori_loop(0, RB, relu_body, 0)
                    c0 = pl.multiple_of(cg * W, 8)
                    pltpu.sync_copy(ybuf, out_h.at[k, pl.ds(r0, RB), pl.ds(c0, W)])
            plsc.subcore_barrier()


_sc_call = functools.partial(
    pl.kernel,
    mesh=plsc.VectorSubcoreMesh(core_axis_name="c", subcore_axis_name="s"),
    out_type=jax.ShapeDtypeStruct((KK, MM, FF), jnp.float32),
    scratch_types=[
        pltpu.VMEM_SHARED((MM, W), jnp.float32),
        pltpu.VMEM((EB,), jnp.int32),
        pltpu.VMEM((EB,), jnp.int32),
        pltpu.VMEM((EB,), jnp.int32),
        pltpu.VMEM((EB,), jnp.float32),
        pltpu.VMEM((EB, W), jnp.float32),
        pltpu.VMEM((RB, W), jnp.float32),
        pltpu.SemaphoreType.DMA,
    ],
)(_sc_body)


@jax.jit
def kernel(x, L_rows, L_cols, L_vals):
    x0 = jnp.transpose(x, (1, 2, 0)).reshape(MM, FF)
    x0f = x0.reshape(MM, NCH, W).transpose(1, 0, 2).reshape(NCH * MM, W)
    out = _sc_call(x0f,
                   L_rows.astype(jnp.int32).reshape(KK * EE),
                   L_cols.astype(jnp.int32).reshape(KK * EE),
                   L_vals.reshape(KK * EE))
    return jnp.transpose(out.reshape(KK, MM, FIN, NB), (3, 1, 2, 0)).reshape(
        NB, MM, FIN * KK)


# speedup vs baseline: 3.6407x; 2.1735x over previous
import functools

import jax
import jax.numpy as jnp
from jax import lax
from jax.experimental import pallas as pl
from jax.experimental.pallas import tpu as pltpu
from jax.experimental.pallas import tpu_sc as plsc

KK = 3
MM = 10000
EE = 320000
FIN = 128
NB = 4
FF = FIN * NB
W = 128
NCH = FF // W
NS = 16
CH_PER_CORE = NCH // 2
NPASS = CH_PER_CORE * KK

EPT = EE // NS
EB = 80
BPH = 2
EH = EB * BPH
NH = EPT // EH
NT = (NH - 1) // 2
SLAB = 8
RB = 80
NRB = MM // RB
RB_ITERS = -(-NRB // NS)
LANES = 16
GPB = EB // LANES
GRP = EH // LANES


def _scale_half(gbuf, ibuf):
    def grp_body(q, c2):
        j = q // GPB
        o = (q % GPB) * LANES
        vv = ibuf[2 * BPH + j, pl.ds(o, LANES)]
        for e16 in range(LANES):
            v = vv[e16]
            row = q * LANES + e16
            for f8 in range(W // LANES):
                sl = pl.ds(f8 * LANES, LANES)
                gbuf[row, sl] = gbuf[row, sl] * v
        return c2
    lax.fori_loop(0, GRP, grp_body, 0)


def _sc_body(x0f, ed_h, out_h,
             y_sp, ibufA, ibufB, cols_adj, rowsA, rowsB,
             gbufA, gbufB, gsem, ssemA, ssemB, isem):
    cid = lax.axis_index("c")
    sid = lax.axis_index("s")

    def fetch_slab(slab_idx, ibuf):
        r0 = pl.multiple_of(slab_idx * SLAB, 8)
        pltpu.async_copy(ed_h.at[pl.ds(r0, SLAB)], ibuf, isem)

    def drain_slab(ibuf):
        pltpu.make_async_copy(ed_h.at[pl.ds(0, SLAB)], ibuf, isem).wait()

    def adj_cols(ibuf, cgM):
        for j in range(BPH):
            for g in range(GPB):
                sl = pl.ds(g * LANES, LANES)
                cols_adj[j, sl] = ibuf[BPH + j, sl].astype(jnp.int32) + cgM

    def copy_rows(ibuf, rbuf):
        for j in range(BPH):
            for g in range(GPB):
                sl = pl.ds(g * LANES, LANES)
                rbuf[j, sl] = ibuf[j, sl].astype(jnp.int32)

    def fire_gathers(gbuf):
        for j in range(BPH):
            pltpu.async_copy(x0f.at[cols_adj.at[j]],
                             gbuf.at[pl.ds(j * EB, EB)], gsem.at[j])

    def wait_gathers(gbuf):
        for j in range(BPH):
            pltpu.make_async_copy(x0f.at[cols_adj.at[j]],
                                  gbuf.at[pl.ds(j * EB, EB)],
                                  gsem.at[j]).wait()

    def fire_scatters(gbuf, rbuf, ssem):
        for j in range(BPH):
            pltpu.async_copy(gbuf.at[pl.ds(j * EB, EB)],
                             y_sp.at[rbuf.at[j]], ssem.at[j], add=True)

    def wait_scatters(gbuf, rbuf, ssem):
        for j in range(BPH):
            pltpu.make_async_copy(gbuf.at[pl.ds(j * EB, EB)],
                                  y_sp.at[rbuf.at[j]], ssem.at[j]).wait()

    def pass_body(p6, carry):
        k = lax.rem(p6, KK)
        ci = p6 // KK
        cg = cid * CH_PER_CORE + ci
        cgM = cg * MM
        sbase = (k * NS + sid) * NH

        for i in range(RB_ITERS):
            bid = sid + NS * i
            @pl.when(bid < NRB)
            def _():
                r0 = pl.multiple_of(bid * RB, 8)
                src0 = pl.multiple_of(cgM + r0, 8)
                pltpu.sync_copy(x0f.at[pl.ds(src0, RB)],
                                gbufA.at[pl.ds(0, RB)])
                pltpu.sync_copy(gbufA.at[pl.ds(0, RB)],
                                y_sp.at[pl.ds(r0, RB)])
        plsc.subcore_barrier()

        fetch_slab(sbase, ibufA)
        drain_slab(ibufA)
        adj_cols(ibufA, cgM)
        fire_gathers(gbufA)

        def two_halves(t, c2):
            fetch_slab(sbase + 2 * t + 1, ibufB)
            wait_gathers(gbufA)
            copy_rows(ibufA, rowsA)
            _scale_half(gbufA, ibufA)
            fire_scatters(gbufA, rowsA, ssemA)
            drain_slab(ibufB)
            adj_cols(ibufB, cgM)
            @pl.when(t > 0)
            def _():
                wait_scatters(gbufB, rowsB, ssemB)
            fire_gathers(gbufB)

            fetch_slab(sbase + 2 * t + 2, ibufA)
            wait_gathers(gbufB)
            copy_rows(ibufB, rowsB)
            _scale_half(gbufB, ibufB)
            fire_scatters(gbufB, rowsB, ssemB)
            drain_slab(ibufA)
            adj_cols(ibufA, cgM)
            wait_scatters(gbufA, rowsA, ssemA)
            fire_gathers(gbufA)
            return c2
        lax.fori_loop(0, NT, two_halves, 0)

        wait_gathers(gbufA)
        copy_rows(ibufA, rowsA)
        _scale_half(gbufA, ibufA)
        fire_scatters(gbufA, rowsA, ssemA)
        wait_scatters(gbufB, rowsB, ssemB)
        wait_scatters(gbufA, rowsA, ssemA)
        plsc.subcore_barrier()

        for i in range(RB_ITERS):
            bid = sid + NS * i
            @pl.when(bid < NRB)
            def _():
                r0 = pl.multiple_of(bid * RB, 8)
                pltpu.sync_copy(y_sp.at[pl.ds(r0, RB)],
                                gbufA.at[pl.ds(0, RB)])

                def relu_body(r, c3):
                    for f8 in range(W // LANES):
                        sl = pl.ds(f8 * LANES, LANES)
                        gbufA[r, sl] = jnp.maximum(gbufA[r, sl], 0.0)
                    return c3
                lax.fori_loop(0, RB, relu_body, 0)
                c0 = pl.multiple_of(cg * W, 8)
                pltpu.sync_copy(gbufA.at[pl.ds(0, RB)],
                                out_h.at[k, pl.ds(r0, RB), pl.ds(c0, W)])
        plsc.subcore_barrier()
        return carry

    lax.fori_loop(0, NPASS, pass_body, 0)


_sc_call = functools.partial(
    pl.kernel,
    mesh=plsc.VectorSubcoreMesh(core_axis_name="c", subcore_axis_name="s"),
    out_type=jax.ShapeDtypeStruct((KK, MM, FF), jnp.float32),
    scratch_types=[
        pltpu.VMEM_SHARED((MM, W), jnp.float32),
        pltpu.VMEM((SLAB, EB), jnp.float32),
        pltpu.VMEM((SLAB, EB), jnp.float32),
        pltpu.VMEM((BPH, EB), jnp.int32),
        pltpu.VMEM((BPH, EB), jnp.int32),
        pltpu.VMEM((BPH, EB), jnp.int32),
        pltpu.VMEM((EH, W), jnp.float32),
        pltpu.VMEM((EH, W), jnp.float32),
        pltpu.SemaphoreType.DMA((BPH,)),
        pltpu.SemaphoreType.DMA((BPH,)),
        pltpu.SemaphoreType.DMA((BPH,)),
        pltpu.SemaphoreType.DMA,
    ],
)(_sc_body)


@jax.jit
def kernel(x, L_rows, L_cols, L_vals):
    x0 = jnp.transpose(x, (1, 2, 0)).reshape(MM, FF)
    x0f = x0.reshape(MM, NCH, W).transpose(1, 0, 2).reshape(NCH * MM, W)
    r5 = L_rows.astype(jnp.float32).reshape(KK, NS, NH, BPH, EB)
    c5 = L_cols.astype(jnp.float32).reshape(KK, NS, NH, BPH, EB)
    v5 = L_vals.reshape(KK, NS, NH, BPH, EB)
    ed = jnp.concatenate([r5, c5, v5], axis=3)
    ed = jnp.pad(ed, ((0, 0), (0, 0), (0, 0), (0, SLAB - 3 * BPH), (0, 0)))
    ed = ed.reshape(KK * NS * NH * SLAB, EB)
    out = _sc_call(x0f, ed)
    return jnp.transpose(out.reshape(KK, MM, FIN, NB), (3, 1, 2, 0)).reshape(
        NB, MM, FIN * KK)


# speedup vs baseline: 5.0493x; 1.3869x over previous
import functools

import jax
import jax.numpy as jnp
from jax import lax
from jax.experimental import pallas as pl
from jax.experimental.pallas import tpu as pltpu
from jax.experimental.pallas import tpu_sc as plsc

KK = 3
MM = 10000
EE = 320000
FIN = 128
NB = 4
FF = FIN * NB
W = 128
NCH = FF // W
NS = 16
CH_PER_CORE = NCH // 2
NPASS = CH_PER_CORE * KK

EPT = EE // NS
EH = 80
NH = EPT // EH
NBUF = 4
NT = (NH - 2) // NBUF
SLAB = 8
RB = 80
NRB = MM // RB
RB_ITERS = -(-NRB // NS)
LANES = 16
GRP = EH // LANES


def _sc_body(x0f, ed_h, out_h,
             y_sp, ibufs, cols_adj, rowsb, gbufs, gsem, ssem, isem):
    cid = lax.axis_index("c")
    sid = lax.axis_index("s")

    def make_ops(cgM, sbase):
        def fetch(h, slot):
            r0 = pl.multiple_of((sbase + h) * SLAB, 8)
            pltpu.async_copy(ed_h.at[pl.ds(r0, SLAB)], ibufs.at[slot],
                             isem.at[slot])

        def drain_fetch(slot):
            pltpu.make_async_copy(ed_h.at[pl.ds(0, SLAB)], ibufs.at[slot],
                                  isem.at[slot]).wait()

        def adj(slot):
            for g in range(GRP):
                sl = pl.ds(g * LANES, LANES)
                cols_adj[slot, sl] = ibufs[slot, 1, sl].astype(jnp.int32) + cgM

        def fire_gather(slot):
            pltpu.async_copy(x0f.at[cols_adj.at[slot]], gbufs.at[slot],
                             gsem.at[slot])

        def wait_gather(slot):
            pltpu.make_async_copy(x0f.at[cols_adj.at[slot]], gbufs.at[slot],
                                  gsem.at[slot]).wait()

        def fire_scatter(slot):
            pltpu.async_copy(gbufs.at[slot], y_sp.at[rowsb.at[slot]],
                             ssem.at[slot], add=True)

        def wait_scatter(slot):
            pltpu.make_async_copy(gbufs.at[slot], y_sp.at[rowsb.at[slot]],
                                  ssem.at[slot]).wait()

        def prep(slot):
            drain_fetch(slot)
            adj(slot)

        def process(slot):
            wait_gather(slot)
            for g in range(GRP):
                sl = pl.ds(g * LANES, LANES)
                rowsb[slot, sl] = ibufs[slot, 0, sl].astype(jnp.int32)

            def grp_body(q, c2):
                vv = ibufs[slot, 2, pl.ds(q * LANES, LANES)]
                for e16 in range(LANES):
                    v = vv[e16]
                    row = q * LANES + e16
                    for f8 in range(W // LANES):
                        sl2 = pl.ds(f8 * LANES, LANES)
                        gbufs[slot, row, sl2] = gbufs[slot, row, sl2] * v
                return c2
            lax.fori_loop(0, GRP, grp_body, 0)
            fire_scatter(slot)

        return fetch, prep, fire_gather, wait_scatter, process

    def pass_body(p6, carry):
        k = lax.rem(p6, KK)
        ci = p6 // KK
        cg = cid * CH_PER_CORE + ci
        cgM = cg * MM
        sbase = (k * NS + sid) * NH
        fetch, prep, fire_gather, wait_scatter, process = make_ops(cgM, sbase)

        for i in range(RB_ITERS):
            bid = sid + NS * i
            @pl.when(bid < NRB)
            def _():
                r0 = pl.multiple_of(bid * RB, 8)
                src0 = pl.multiple_of(cgM + r0, 8)
                pltpu.sync_copy(x0f.at[pl.ds(src0, RB)], gbufs.at[0])
                pltpu.sync_copy(gbufs.at[0], y_sp.at[pl.ds(r0, RB)])
        plsc.subcore_barrier()

        fetch(0, 0)
        fetch(1, 1)
        fetch(2, 2)
        prep(0)
        fire_gather(0)
        prep(1)
        fire_gather(1)
        fetch(3, 3)
        prep(2)
        fire_gather(2)
        process(0)
        fetch(4, 0)
        prep(3)
        fire_gather(3)
        process(1)

        def steady(u, c2):
            for p in range(NBUF):
                g = 2 + NBUF * u + p
                s_cur = (2 + p) % NBUF
                s_nxt2 = p
                s_nxt3 = (p + 1) % NBUF
                if p == 0:
                    fetch(g + 3, s_nxt3)
                else:
                    @pl.when(u < NT - 1)
                    def _():
                        fetch(g + 3, s_nxt3)
                if p < 2:
                    prep(s_nxt2)
                    wait_scatter(s_nxt2)
                    fire_gather(s_nxt2)
                else:
                    @pl.when(u < NT - 1)
                    def _():
                        prep(s_nxt2)
                        wait_scatter(s_nxt2)
                        fire_gather(s_nxt2)
                process(s_cur)
            return c2
        lax.fori_loop(0, NT, steady, 0)

        for slot in range(NBUF):
            wait_scatter(slot)
        plsc.subcore_barrier()

        for i in range(RB_ITERS):
            bid = sid + NS * i
            @pl.when(bid < NRB)
            def _():
                r0 = pl.multiple_of(bid * RB, 8)
                pltpu.sync_copy(y_sp.at[pl.ds(r0, RB)], gbufs.at[0])

                def relu_body(r, c3):
                    for f8 in range(W // LANES):
                        sl = pl.ds(f8 * LANES, LANES)
                        gbufs[0, r, sl] = jnp.maximum(gbufs[0, r, sl], 0.0)
                    return c3
                lax.fori_loop(0, RB, relu_body, 0)
                c0 = pl.multiple_of(cg * W, 8)
                pltpu.sync_copy(gbufs.at[0],
                                out_h.at[k, pl.ds(r0, RB), pl.ds(c0, W)])
        plsc.subcore_barrier()
        return carry

    lax.fori_loop(0, NPASS, pass_body, 0)


_sc_call = functools.partial(
    pl.kernel,
    mesh=plsc.VectorSubcoreMesh(core_axis_name="c", subcore_axis_name="s"),
    out_type=jax.ShapeDtypeStruct((KK, MM, FF), jnp.float32),
    scratch_types=[
        pltpu.VMEM_SHARED((MM, W), jnp.float32),
        pltpu.VMEM((NBUF, SLAB, EH), jnp.float32),
        pltpu.VMEM((NBUF, EH), jnp.int32),
        pltpu.VMEM((NBUF, EH), jnp.int32),
        pltpu.VMEM((NBUF, EH, W), jnp.float32),
        pltpu.SemaphoreType.DMA((NBUF,)),
        pltpu.SemaphoreType.DMA((NBUF,)),
        pltpu.SemaphoreType.DMA((NBUF,)),
    ],
)(_sc_body)


@jax.jit
def kernel(x, L_rows, L_cols, L_vals):
    x0 = jnp.transpose(x, (1, 2, 0)).reshape(MM, FF)
    x0f = x0.reshape(MM, NCH, W).transpose(1, 0, 2).reshape(NCH * MM, W)
    r5 = L_rows.astype(jnp.float32).reshape(KK, NS, NH, 1, EH)
    c5 = L_cols.astype(jnp.float32).reshape(KK, NS, NH, 1, EH)
    v5 = L_vals.reshape(KK, NS, NH, 1, EH)
    ed = jnp.concatenate([r5, c5, v5], axis=3)
    ed = jnp.pad(ed, ((0, 0), (0, 0), (0, 0), (0, SLAB - 3), (0, 0)))
    ed = ed.reshape(KK * NS * NH * SLAB, EH)
    out = _sc_call(x0f, ed)
    return jnp.transpose(out.reshape(KK, MM, FIN, NB), (3, 1, 2, 0)).reshape(
        NB, MM, FIN * KK)
